# Initial kernel scaffold; baseline (speedup 1.0000x reference)
import jax
import jax.numpy as jnp
from jax.experimental import pallas as pl


def kernel(x, params):
    raise NotImplementedError("write your pallas kernel here")


# speedup vs baseline: 7.2014x; 7.2014x over previous
import functools

import jax
import jax.numpy as jnp
from jax.experimental import pallas as pl
from jax.experimental.pallas import tpu as pltpu
from jax.experimental.pallas import tpu_sc as plsc

F32 = jnp.float32
I32 = jnp.int32


def _dot(a, b):
    return jnp.dot(a, b, preferred_element_type=F32)
DM = 256


def _pad_lanes(x, w):
    if x.shape[-1] == w:
        return x
    return jnp.pad(x, [(0, 0)] * (x.ndim - 1) + [(0, w - x.shape[-1])])


def _sc_gather(table, flat_idx):
    rg = flat_idx.shape[0]
    d = table.shape[1]
    w = 128
    rgp = ((rg + w - 1) // w) * w
    if rgp != rg:
        flat_idx = jnp.pad(flat_idx, (0, rgp - rg))
    idx2 = flat_idx.reshape(1, rgp)
    mesh = plsc.VectorSubcoreMesh(core_axis_name="c", subcore_axis_name="s")

    @functools.partial(
        pl.kernel,
        out_type=jax.ShapeDtypeStruct((rgp, d), table.dtype),
        mesh=mesh,
    )
    def gather_kernel(tab_hbm, i_hbm, o_hbm):
        def body(i_vmem, o_vmem):
            pltpu.sync_copy(tab_hbm.at[i_vmem.at[0]], o_vmem)

        pltpu.emit_pipeline(
            body,
            grid=(rgp // w,),
            in_specs=[pl.BlockSpec((1, w), lambda i: (0, i))],
            out_specs=[pl.BlockSpec((w, d), lambda i: (i, 0))],
            core_axis_name=("c", "s"),
            dimension_semantics=(pltpu.PARALLEL,),
        )(i_hbm, o_hbm)

    out = gather_kernel(table, idx2)
    return out if rgp == rg else out[:rg]


def _fps(xyzT, npoint):
    _, b, n = xyzT.shape

    def body(x_ref, out_ref):
        lane = jax.lax.broadcasted_iota(I32, (1, n), 1)
        np_lane = jax.lax.broadcasted_iota(I32, (1, npoint), 1)
        rows = [(x_ref[0, j:j + 1, :], x_ref[1, j:j + 1, :],
                 x_ref[2, j:j + 1, :]) for j in range(b)]

        def step(i, carry):
            out = []
            for j in range(b):
                dist, acc = carry[2 * j], carry[2 * j + 1]
                xr, yr, zr = rows[j]
                mx = jnp.max(dist)
                far = jnp.min(jnp.where(dist == mx, lane, n))
                acc = jnp.where(np_lane == i, far, acc)
                mask = lane == far
                cx = jnp.sum(jnp.where(mask, xr, 0.0))
                cy = jnp.sum(jnp.where(mask, yr, 0.0))
                cz = jnp.sum(jnp.where(mask, zr, 0.0))
                dx = xr - cx
                dy = yr - cy
                dz = zr - cz
                d = dx * dx + dy * dy + dz * dz
                out.append(jnp.minimum(dist, d))
                out.append(acc)
            return tuple(out)

        init = []
        for j in range(b):
            init.append(jnp.full((1, n), 1e10, F32))
            init.append(jnp.zeros((1, npoint), I32))
        res = jax.lax.fori_loop(0, npoint, step, tuple(init))
        for j in range(b):
            out_ref[j:j + 1, :] = res[2 * j + 1] + j * n

    return pl.pallas_call(
        body,
        out_shape=jax.ShapeDtypeStruct((b, npoint), I32),
    )(xyzT)


def _knn(src, dstT, k, kpad):
    bb, m, _ = src.shape
    n = dstT.shape[2]
    tile = min(128, m)

    def body(s_ref, dT_ref, idx_ref, dst_ref):
        b = pl.program_id(0)
        s = s_ref[0]
        dT = dT_ref[0]
        s2 = jnp.sum(s * s, -1, keepdims=True)
        xg, yg, zg = dT[0:1, :], dT[1:2, :], dT[2:3, :]
        d2 = xg * xg + yg * yg + zg * zg
        cross = jnp.dot(s, dT, preferred_element_type=F32)
        dist = s2 + d2 - 2.0 * cross
        lane = jax.lax.broadcasted_iota(I32, (tile, n), 1)
        idxs, dsts = [], []
        for _ in range(k):
            mval = jnp.min(dist, -1, keepdims=True)
            amin = jnp.min(jnp.where(dist == mval, lane, n), -1, keepdims=True)
            idxs.append(amin)
            dsts.append(mval)
            dist = jnp.where(lane == amin, F32(3e38), dist)
        for _ in range(kpad - k):
            idxs.append(jnp.zeros((tile, 1), I32))
            dsts.append(jnp.zeros((tile, 1), F32))
        idx_ref[0] = jnp.concatenate(idxs, -1) + b * n
        dst_ref[0] = jnp.concatenate(dsts, -1)

    return pl.pallas_call(
        body,
        grid=(bb, m // tile),
        in_specs=[
            pl.BlockSpec((1, tile, 3), lambda b, i: (b, i, 0)),
            pl.BlockSpec((1, 3, n), lambda b, i: (b, 0, 0)),
        ],
        out_specs=[
            pl.BlockSpec((1, tile, kpad), lambda b, i: (b, i, 0)),
            pl.BlockSpec((1, tile, kpad), lambda b, i: (b, i, 0)),
        ],
        out_shape=[
            jax.ShapeDtypeStruct((bb, m, kpad), I32),
            jax.ShapeDtypeStruct((bb, m, kpad), F32),
        ],
    )(src, dstT)


def _mlp(x, layers):
    r = x.shape[0]
    tile = min(512, r)
    nlayers = len(layers)
    relus = tuple(l[2] for l in layers)

    def body(x_ref, *refs):
        out_ref = refs[-1]
        h = x_ref[...]
        for i in range(nlayers):
            w = refs[2 * i][...]
            bvec = refs[2 * i + 1]
            h = _dot(h, w)
            if bvec is not None:
                h = h + bvec[...]
            if relus[i]:
                h = jnp.maximum(h, 0.0)
        out_ref[...] = h

    args = [x]
    in_specs = [pl.BlockSpec((tile, x.shape[1]), lambda i: (i, 0))]
    for wT, bvec, _ in layers:
        args.append(wT)
        in_specs.append(pl.BlockSpec(wT.shape, lambda i: (0, 0)))
        args.append(bvec.reshape(1, -1) if bvec is not None else None)
        in_specs.append(
            pl.BlockSpec((1, wT.shape[1]), lambda i: (0, 0))
            if bvec is not None else None
        )
    body_args, body_specs = [], []
    keep = [a is not None for a in args]

    def body2(*refs):
        it = iter(refs[:-1])
        full = [next(it) if kp else None for kp in keep]
        out_ref = refs[-1]
        h = full[0][...]
        for i in range(nlayers):
            w = full[1 + 2 * i][...]
            bref = full[2 + 2 * i]
            h = _dot(h, w)
            if bref is not None:
                h = h + bref[...]
            if relus[i]:
                h = jnp.maximum(h, 0.0)
        out_ref[...] = h

    for a, sp in zip(args, in_specs):
        if a is not None:
            body_args.append(a)
            body_specs.append(sp)

    dout = layers[-1][0].shape[1]
    return pl.pallas_call(
        body2,
        grid=(r // tile,),
        in_specs=body_specs,
        out_specs=pl.BlockSpec((tile, dout), lambda i: (i, 0)),
        out_shape=jax.ShapeDtypeStruct((r, dout), F32),
    )(*body_args)


def _lin_args(p):
    return p["w"].T, p["b"]


def _proj(feats, p):
    r, d = feats.shape
    tile = min(512, r)
    w1T, b1 = p["fc1"]["w"].T, p["fc1"]["b"].reshape(1, -1)
    wqT = p["w_qs"]["w"].T
    wkT = p["w_ks"]["w"].T
    wvT = p["w_vs"]["w"].T

    def body(f_ref, w1_ref, b1_ref, wq_ref, wk_ref, wv_ref,
             q_ref, k_ref, v_ref):
        x1 = _dot(f_ref[...], w1_ref[...])
        x1 = x1 + b1_ref[...]
        q_ref[...] = _dot(x1, wq_ref[...])
        k_ref[...] = _dot(x1, wk_ref[...])
        v_ref[...] = _dot(x1, wv_ref[...])

    full = lambda a: pl.BlockSpec(a.shape, lambda i: (0, 0))
    rowspec = pl.BlockSpec((tile, DM), lambda i: (i, 0))
    return pl.pallas_call(
        body,
        grid=(r // tile,),
        in_specs=[
            pl.BlockSpec((tile, d), lambda i: (i, 0)),
            full(w1T), full(b1), full(wqT), full(wkT), full(wvT),
        ],
        out_specs=[rowspec, rowspec, rowspec],
        out_shape=[jax.ShapeDtypeStruct((r, DM), F32)] * 3,
    )(feats, w1T, b1, wqT, wkT, wvT)


def _attention(q, xyzp, pre, gkf, gv, gx, p, keff):
    r, d = pre.shape
    tile = min(128, r)
    d1T = _pad_lanes(p["fc_delta"][0]["w"], 16).T
    d1b = p["fc_delta"][0]["b"].reshape(1, -1)
    d2T = p["fc_delta"][1]["w"].T
    d2b = p["fc_delta"][1]["b"].reshape(1, -1)
    g1T = p["fc_gamma"][0]["w"].T
    g1b = p["fc_gamma"][0]["b"].reshape(1, -1)
    g2T = p["fc_gamma"][1]["w"].T
    g2b = p["fc_gamma"][1]["b"].reshape(1, -1)
    w2T = p["fc2"]["w"].T
    w2b = p["fc2"]["b"].reshape(1, -1)

    def body(q_ref, xp_ref, pre_ref, gkf_ref, gv_ref, gx_ref, d1_ref,
             d1b_ref, d2_ref, d2b_ref, g1_ref, g1b_ref, g2_ref, g2b_ref,
             w2_ref, w2b_ref, out_ref):
        kf = gkf_ref[...]
        xg = gx_ref[...][:, 0:16]
        vv = gv_ref[...]
        xq = xp_ref[...]
        xq3 = jnp.broadcast_to(
            xq[:, None, :], (tile, keff, 16)).reshape(tile * keff, 16)
        dx = xq3 - xg
        h = _dot(dx, d1_ref[...]) + d1b_ref[...]
        delta = _dot(jnp.maximum(h, 0.0), d2_ref[...]) + d2b_ref[...]
        q3 = jnp.broadcast_to(
            q_ref[...][:, None, :], (tile, keff, DM)).reshape(tile * keff, DM)
        u = q3 - kf + delta
        h = _dot(u, g1_ref[...]) + g1b_ref[...]
        attn = _dot(jnp.maximum(h, 0.0), g2_ref[...]) + g2b_ref[...]
        a3 = (attn / 16.0).reshape(tile, keff, DM)
        mx = jnp.max(a3, axis=1, keepdims=True)
        e = jnp.exp(a3 - mx)
        sm = e / jnp.sum(e, axis=1, keepdims=True)
        w3 = (vv + delta).reshape(tile, keff, DM)
        res = jnp.sum(sm * w3, axis=1)
        out_ref[...] = (
            _dot(res, w2_ref[...])
            + w2b_ref[...] + pre_ref[...]
        )

    full = lambda a: pl.BlockSpec(a.shape, lambda i: (0, 0))
    return pl.pallas_call(
        body,
        grid=(r // tile,),
        in_specs=[
            pl.BlockSpec((tile, DM), lambda i: (i, 0)),
            pl.BlockSpec((tile, 16), lambda i: (i, 0)),
            pl.BlockSpec((tile, d), lambda i: (i, 0)),
            pl.BlockSpec((tile * keff, DM), lambda i: (i, 0)),
            pl.BlockSpec((tile * keff, DM), lambda i: (i, 0)),
            pl.BlockSpec((tile * keff, 128), lambda i: (i, 0)),
            full(d1T), full(d1b), full(d2T), full(d2b),
            full(g1T), full(g1b), full(g2T), full(g2b),
            full(w2T), full(w2b),
        ],
        out_specs=pl.BlockSpec((tile, d), lambda i: (i, 0)),
        out_shape=jax.ShapeDtypeStruct((r, d), F32),
    )(q, xyzp, pre, gkf, gv, gx, d1T, d1b, d2T, d2b, g1T, g1b, g2T, g2b,
      w2T, w2b)


def _transformer(p, feats, xyzp, gx, knn_idx_flat, keff):
    q, kf, v = _proj(feats, p)
    gkf = _sc_gather(kf, knn_idx_flat)
    gv = _sc_gather(v, knn_idx_flat)
    return _attention(q, xyzp, pre=feats, gkf=gkf, gv=gv, gx=gx, p=p,
                      keff=keff)


def _transition_down(gath, nxp, p, keff, din):
    rk, wp = gath.shape
    r = rk // keff
    cin = 3 + din
    w1T = p["convs"][0]["lin"]["w"].T
    b1 = p["convs"][0]["lin"]["b"].reshape(1, -1)
    g1 = p["convs"][0]["bn"]["gamma"].reshape(1, -1)
    be1 = p["convs"][0]["bn"]["beta"].reshape(1, -1)
    w2T = p["convs"][1]["lin"]["w"].T
    b2 = p["convs"][1]["lin"]["b"].reshape(1, -1)
    g2 = p["convs"][1]["bn"]["gamma"].reshape(1, -1)
    be2 = p["convs"][1]["bn"]["beta"].reshape(1, -1)
    ch = w1T.shape[1]

    def bn_relu(h, gam, bet):
        mean = jnp.mean(h, axis=0, keepdims=True)
        var = jnp.mean((h - mean) ** 2, axis=0, keepdims=True)
        return jnp.maximum(
            (h - mean) / jnp.sqrt(var + 1e-5) * gam + bet, 0.0)

    def body(g_ref, nx_ref, w1_ref, b1_ref, g1_ref, be1_ref,
             w2_ref, b2_ref, g2_ref, be2_ref, out_ref):
        g = g_ref[...]
        nx = nx_ref[...][:, 0:3]
        nx3 = jnp.broadcast_to(
            nx[:, None, :], (r, keff, 3)).reshape(rk, 3)
        u = jnp.concatenate([g[:, 0:3] - nx3, g[:, 3:3 + din]], axis=-1)
        h = _dot(u, w1_ref[...]) + b1_ref[...]
        h = bn_relu(h, g1_ref[...], be1_ref[...])
        h = _dot(h, w2_ref[...]) + b2_ref[...]
        h = bn_relu(h, g2_ref[...], be2_ref[...])
        out_ref[...] = jnp.max(h.reshape(r, keff, ch), axis=1)

    return pl.pallas_call(
        body,
        out_shape=jax.ShapeDtypeStruct((r, ch), F32),
    )(gath, nxp, w1T, b1, g1, be1, w2T, b2, g2, be2)


def _bn_mlp(x, p):
    r, din = x.shape
    wT = p["lin"]["w"].T
    b = p["lin"]["b"].reshape(1, -1)
    gam = p["bn"]["gamma"].reshape(1, -1)
    bet = p["bn"]["beta"].reshape(1, -1)
    ch = wT.shape[1]

    def body(x_ref, w_ref, b_ref, g_ref, be_ref, out_ref):
        h = _dot(x_ref[...], w_ref[...]) + b_ref[...]
        mean = jnp.mean(h, axis=0, keepdims=True)
        var = jnp.mean((h - mean) ** 2, axis=0, keepdims=True)
        out_ref[...] = jnp.maximum(
            (h - mean) / jnp.sqrt(var + 1e-5) * g_ref[...] + be_ref[...], 0.0)

    return pl.pallas_call(
        body,
        out_shape=jax.ShapeDtypeStruct((r, ch), F32),
    )(x, wT, b, gam, bet)


def _feature_prop(g0, g1, g2, d3, feats2):
    r, ch = feats2.shape

    def body(g0_ref, g1_ref, g2_ref, d_ref, f2_ref, out_ref):
        d = d_ref[...]
        w0 = 1.0 / (d[:, 0:1] + 1e-8)
        w1 = 1.0 / (d[:, 1:2] + 1e-8)
        w2 = 1.0 / (d[:, 2:3] + 1e-8)
        ws = (w0 + w1) + w2
        out_ref[...] = (
            g0_ref[...][:, 0:ch] * (w0 / ws) + g1_ref[...][:, 0:ch] * (w1 / ws)
            + g2_ref[...][:, 0:ch] * (w2 / ws) + f2_ref[...]
        )

    return pl.pallas_call(
        body,
        out_shape=jax.ShapeDtypeStruct((r, ch), F32),
    )(g0, g1, g2, d3, feats2)


def kernel(x, params):
    bb, n0, _ = x.shape
    nn = 16

    xyz0 = x[..., 0:3]

    ns = [n0, n0 // 4, n0 // 16, n0 // 64, n0 // 256]
    xyz = [xyz0]
    fps_flat = []
    for l in range(1, 5):
        prev = xyz[-1]
        m = ns[l]
        xyzT = jnp.transpose(prev, (2, 0, 1))
        sidx = _fps(xyzT, m)
        flat = sidx.reshape(bb * m)
        fps_flat.append(flat)
        xyz_tab = _pad_lanes(prev.reshape(bb * ns[l - 1], 3), 128)
        nxp = _sc_gather(xyz_tab, flat)
        xyz.append(nxp[:, 0:3].reshape(bb, m, 3))

    xyzT_l = [jnp.transpose(z, (0, 2, 1)) for z in xyz]
    xyzp_l = [_pad_lanes(z.reshape(-1, 3), 16) for z in xyz]
    xt128_l = [_pad_lanes(z.reshape(-1, 3), 128) for z in xyz]

    keff_l = [min(nn, ns[l]) for l in range(5)]
    knn_self = []
    gx_self = []
    for l in range(5):
        idx, _ = _knn(xyz[l], xyzT_l[l], keff_l[l], keff_l[l])
        flat = idx.reshape(-1)
        knn_self.append(flat)
        gx_self.append(_sc_gather(xt128_l[l], flat))
    knn_down = []
    for i in range(4):
        idx, _ = _knn(xyz[i + 1], xyzT_l[i], nn, nn)
        knn_down.append(idx.reshape(-1))
    knn_up = []
    for i in range(4):
        fine, coarse = 3 - i, 4 - i
        idx, dst = _knn(xyz[fine], xyzT_l[coarse], 3, 4)
        knn_up.append((idx, dst.reshape(-1, 4)))

    feats = _mlp(
        x.reshape(bb * n0, x.shape[-1]),
        [(params["fc1"][0]["w"].T, params["fc1"][0]["b"], True),
         (params["fc1"][1]["w"].T, params["fc1"][1]["b"], False)],
    )
    points = _transformer(
        params["transformer1"], feats, xyzp_l[0], gx_self[0], knn_self[0],
        keff_l[0])

    level_points = [points]
    for i in range(4):
        din = level_points[-1].shape[1]
        tab = jnp.concatenate(
            [xyz[i].reshape(-1, 3), level_points[-1]], axis=-1)
        tab = _pad_lanes(tab, ((3 + din + 127) // 128) * 128)
        gath = _sc_gather(tab, knn_down[i])
        pts = _transition_down(
            gath, xyzp_l[i + 1], params["td"][i], nn, din)
        pts = _transformer(
            params["tr"][i], pts, xyzp_l[i + 1], gx_self[i + 1],
            knn_self[i + 1], keff_l[i + 1])
        level_points.append(pts)

    h = _mlp(
        level_points[4],
        [(params["fc2"][0]["w"].T, params["fc2"][0]["b"], True),
         (params["fc2"][1]["w"].T, params["fc2"][1]["b"], True),
         (params["fc2"][2]["w"].T, params["fc2"][2]["b"], False)],
    )
    points = _transformer(
        params["transformer2"], h, xyzp_l[4], gx_self[4], knn_self[4],
        keff_l[4])

    for i in range(4):
        fine = 3 - i
        feats1 = _bn_mlp(points, params["tu"][i]["fc1"])
        feats2 = _bn_mlp(level_points[fine], params["tu"][i]["fc2"])
        ch = feats1.shape[1]
        f1p = _pad_lanes(feats1, max(128, ch))
        idx, d3 = knn_up[i]
        g = [_sc_gather(f1p, idx[:, :, j].reshape(-1)) for j in range(3)]
        points = _feature_prop(g[0], g[1], g[2], d3, feats2)
        points = _transformer(
            params["trs"][i], points, xyzp_l[fine], gx_self[fine],
            knn_self[fine], keff_l[fine])

    out = _mlp(
        points,
        [(params["fc3"][0]["w"].T, params["fc3"][0]["b"], True),
         (params["fc3"][1]["w"].T, params["fc3"][1]["b"], True),
         (params["fc3"][2]["w"].T, params["fc3"][2]["b"], False)],
    )
    return out.reshape(bb, n0, -1)
